# baseline (speedup 1.0000x reference)
import functools

import jax
import jax.numpy as jnp
import numpy as np
from jax import lax
from jax.experimental import pallas as pl
from jax.experimental.pallas import tpu as pltpu
from jax.experimental.pallas import tpu_sc as plsc

_FIELD_DIMS = [100000] * 26
_EMBED_DIM = 64
_OFFS = np.array((0, *np.cumsum(_FIELD_DIMS)[:-1]), dtype=np.int32)

_NC = 2
_NS = 16
_NW = _NC * _NS
_LANES = 16
_CHUNK = 128
_CONV_COLS = 2048


_HALF = _CONV_COLS // 2


def _conv_body(tt_ref, eye_ref, out_ref):
    t = lax.dot_general(tt_ref[...], eye_ref[...],
                        (((0,), (0,)), ((), ())),
                        preferred_element_type=jnp.float32)
    out_ref[...] = jnp.concatenate([t[:_HALF], t[_HALF:]], axis=1)


@functools.lru_cache(maxsize=None)
def _conv_tc(V, D):
    grid = (V + _CONV_COLS - 1) // _CONV_COLS
    return pl.pallas_call(
        _conv_body,
        grid=(grid,),
        in_specs=[
            pl.BlockSpec((D, _CONV_COLS), lambda i: (0, i)),
            pl.BlockSpec((D, D), lambda i: (0, 0)),
        ],
        out_specs=pl.BlockSpec((_HALF, 2 * D), lambda i: (i, 0)),
        out_shape=jax.ShapeDtypeStruct((grid * _HALF, 2 * D), jnp.float32),
    )


@functools.lru_cache(maxsize=None)
def _sc_gather(B, nchunk):
    mesh = plsc.VectorSubcoreMesh(core_axis_name="c", subcore_axis_name="s")
    bpw = nchunk * _CHUNK

    @functools.partial(
        pl.kernel,
        mesh=mesh,
        out_type=jax.ShapeDtypeStruct((B // 2, 128), jnp.float32),
        scratch_types=[
            pltpu.VMEM((nchunk, _CHUNK), jnp.int32),
            pltpu.VMEM((nchunk, _CHUNK), jnp.int32),
            pltpu.VMEM((nchunk, _CHUNK), jnp.int32),
            pltpu.VMEM((nchunk, _CHUNK), jnp.int32),
            pltpu.VMEM((_CHUNK, 128), jnp.float32),
            pltpu.VMEM((_CHUNK, 128), jnp.float32),
            pltpu.VMEM((_CHUNK // 2, 128), jnp.float32),
            pltpu.SemaphoreType.DMA,
            pltpu.SemaphoreType.DMA,
            pltpu.SemaphoreType.DMA,
        ],
        compiler_params=pltpu.CompilerParams(needs_layout_passes=False),
    )
    def k(x_hbm, off_hbm, t2_hbm, out_hbm,
          x_v, off_v, row_v, sel_v, gb0, gb1, stage, g0, g1, osem):
        wid = lax.axis_index("s") * _NC + lax.axis_index("c")

        pltpu.sync_copy(x_hbm.at[wid], x_v)
        pltpu.sync_copy(off_hbm, off_v)

        def idx_body(c, carry):
            for j in range(_CHUNK // _LANES):
                sl = pl.ds(j * _LANES, _LANES)
                idx = x_v[c, sl] + off_v[c, sl]
                row_v[c, sl] = ((idx >> 11) << 10) | (idx & (_HALF - 1))
                sel_v[c, sl] = ((idx >> 10) & 1) << 6
            return carry

        lax.fori_loop(0, nchunk, idx_body, 0)

        gbufs, gsems = (gb0, gb1), (g0, g1)

        def start_gather(c, b):
            pltpu.async_copy(t2_hbm.at[row_v.at[c]], gbufs[b], gsems[b])

        def wait_gather(c, b):
            pltpu.make_async_copy(t2_hbm.at[row_v.at[c]], gbufs[b],
                                  gsems[b]).wait()

        def out_slice(c):
            return out_hbm.at[pl.ds(wid * (bpw // 2) + c * (_CHUNK // 2),
                                    _CHUNK // 2)]

        iota = lax.iota(jnp.int32, _LANES)

        def select_chunk(c, b):
            gbuf = gbufs[b]

            def sel_body(kk, carry):
                selv = sel_v[c, pl.ds(kk * _LANES, _LANES)]
                for j in range(_LANES):
                    p = kk * _LANES + j
                    rowsplat = jnp.full((_LANES,), p, jnp.int32)
                    colbase = jnp.take(selv, jnp.full((_LANES,), j, jnp.int32))
                    for m in range(_EMBED_DIM // _LANES):
                        vals = plsc.load_gather(
                            gbuf, [rowsplat, colbase + (iota + m * _LANES)])
                        stage[p >> 1,
                              pl.ds((p & 1) * _EMBED_DIM + m * _LANES,
                                    _LANES)] = vals
                return carry

            lax.fori_loop(0, _CHUNK // _LANES, sel_body, 0)

        start_gather(0, 0)
        start_gather(1, 1)

        def pipe_body(h, carry):
            c = h * 2
            for b in range(2):
                cc = c + b
                wait_gather(cc, b)
                select_chunk(cc, b)
                start_gather(cc + 2, b)
                pltpu.sync_copy(stage, out_slice(cc))
            return carry

        lax.fori_loop(0, (nchunk - 2) // 2, pipe_body, 0)

        for b in range(2):
            cc = nchunk - 2 + b
            wait_gather(cc, b)
            select_chunk(cc, b)
            pltpu.sync_copy(stage, out_slice(cc))

    return k


def kernel(x, table):
    batch, nf = x.shape
    V, D = table.shape
    B = batch * nf
    bpw = B // _NW
    nchunk = bpw // _CHUNK
    assert bpw % _CHUNK == 0 and bpw % nf == 0 and nchunk % 2 == 0
    assert D == _EMBED_DIM and V % 2 == 0

    t2 = _conv_tc(V, D)(table.T, jnp.eye(D, dtype=jnp.float32))
    x2 = x.reshape(_NW, nchunk, _CHUNK)
    off_flat = np.tile(_OFFS, bpw // nf).reshape(nchunk, _CHUNK)
    out = _sc_gather(B, nchunk)(x2, jnp.asarray(off_flat), t2)
    return out.reshape(batch, nf, D)


# speedup vs baseline: 1.8541x; 1.8541x over previous
import functools

import jax
import jax.numpy as jnp
import numpy as np
from jax import lax
from jax.experimental import pallas as pl
from jax.experimental.pallas import tpu as pltpu
from jax.experimental.pallas import tpu_sc as plsc

_FIELD_DIMS = [100000] * 26
_EMBED_DIM = 64
_OFFS = np.array((0, *np.cumsum(_FIELD_DIMS)[:-1]), dtype=np.int32)

_NC = 2
_NS = 16
_NW = _NC * _NS
_LANES = 16
_CHUNK = 128
_CONV_COLS = 16384


_HALF = _CONV_COLS // 2
_SH = _CONV_COLS.bit_length() - 1


def _conv_body(tt_ref, eye_ref, out_ref):
    t = lax.dot_general(tt_ref[...], eye_ref[...],
                        (((0,), (0,)), ((), ())),
                        preferred_element_type=jnp.float32)
    out_ref[...] = jnp.concatenate([t[:_HALF], t[_HALF:]], axis=1)


@functools.lru_cache(maxsize=None)
def _conv_tc(V, D):
    grid = (V + _CONV_COLS - 1) // _CONV_COLS
    return pl.pallas_call(
        _conv_body,
        grid=(grid,),
        in_specs=[
            pl.BlockSpec((D, _CONV_COLS), lambda i: (0, i)),
            pl.BlockSpec((D, D), lambda i: (0, 0)),
        ],
        out_specs=pl.BlockSpec((_HALF, 2 * D), lambda i: (i, 0)),
        out_shape=jax.ShapeDtypeStruct((grid * _HALF, 2 * D), jnp.float32),
    )


@functools.lru_cache(maxsize=None)
def _sc_gather(B, nchunk):
    mesh = plsc.VectorSubcoreMesh(core_axis_name="c", subcore_axis_name="s")
    bpw = nchunk * _CHUNK

    @functools.partial(
        pl.kernel,
        mesh=mesh,
        out_type=jax.ShapeDtypeStruct((B // 2, 128), jnp.float32),
        scratch_types=[
            pltpu.VMEM((nchunk, _CHUNK), jnp.int32),
            pltpu.VMEM((nchunk, _CHUNK), jnp.int32),
            pltpu.VMEM((nchunk, _CHUNK), jnp.int32),
            pltpu.VMEM((nchunk, _CHUNK), jnp.int32),
            pltpu.VMEM((_CHUNK, 128), jnp.float32),
            pltpu.VMEM((_CHUNK, 128), jnp.float32),
            pltpu.VMEM((_CHUNK // 2, 128), jnp.float32),
            pltpu.SemaphoreType.DMA,
            pltpu.SemaphoreType.DMA,
            pltpu.SemaphoreType.DMA,
        ],
        compiler_params=pltpu.CompilerParams(needs_layout_passes=False),
    )
    def k(x_hbm, off_hbm, t2_hbm, out_hbm,
          x_v, off_v, row_v, sel_v, gb0, gb1, stage, g0, g1, osem):
        wid = lax.axis_index("s") * _NC + lax.axis_index("c")

        pltpu.sync_copy(x_hbm.at[wid], x_v)
        pltpu.sync_copy(off_hbm, off_v)

        def idx_body(c, carry):
            for j in range(_CHUNK // _LANES):
                sl = pl.ds(j * _LANES, _LANES)
                idx = x_v[c, sl] + off_v[c, sl]
                row_v[c, sl] = ((idx >> _SH) << (_SH - 1)) | (idx & (_HALF - 1))
                sel_v[c, sl] = ((idx >> (_SH - 1)) & 1) << 6
            return carry

        lax.fori_loop(0, nchunk, idx_body, 0)

        gbufs, gsems = (gb0, gb1), (g0, g1)

        def start_gather(c, b):
            pltpu.async_copy(t2_hbm.at[row_v.at[c]], gbufs[b], gsems[b])

        def wait_gather(c, b):
            pltpu.make_async_copy(t2_hbm.at[row_v.at[c]], gbufs[b],
                                  gsems[b]).wait()

        def out_slice(c):
            return out_hbm.at[pl.ds(wid * (bpw // 2) + c * (_CHUNK // 2),
                                    _CHUNK // 2)]

        iota = lax.iota(jnp.int32, _LANES)

        def select_chunk(c, b):
            gbuf = gbufs[b]

            def sel_body(kk, carry):
                selv = sel_v[c, pl.ds(kk * _LANES, _LANES)]
                for j in range(_LANES):
                    p = kk * _LANES + j
                    rowsplat = jnp.full((_LANES,), p, jnp.int32)
                    colbase = jnp.take(selv, jnp.full((_LANES,), j, jnp.int32))
                    for m in range(_EMBED_DIM // _LANES):
                        vals = plsc.load_gather(
                            gbuf, [rowsplat, colbase + (iota + m * _LANES)])
                        stage[p >> 1,
                              pl.ds((p & 1) * _EMBED_DIM + m * _LANES,
                                    _LANES)] = vals
                return carry

            lax.fori_loop(0, _CHUNK // _LANES, sel_body, 0)

        start_gather(0, 0)
        start_gather(1, 1)

        def pipe_body(h, carry):
            c = h * 2
            for b in range(2):
                cc = c + b
                wait_gather(cc, b)
                select_chunk(cc, b)
                start_gather(cc + 2, b)
                pltpu.sync_copy(stage, out_slice(cc))
            return carry

        lax.fori_loop(0, (nchunk - 2) // 2, pipe_body, 0)

        for b in range(2):
            cc = nchunk - 2 + b
            wait_gather(cc, b)
            select_chunk(cc, b)
            pltpu.sync_copy(stage, out_slice(cc))

    return k


def kernel(x, table):
    batch, nf = x.shape
    V, D = table.shape
    B = batch * nf
    bpw = B // _NW
    nchunk = bpw // _CHUNK
    assert bpw % _CHUNK == 0 and bpw % nf == 0 and nchunk % 2 == 0
    assert D == _EMBED_DIM and V % 2 == 0

    t2 = _conv_tc(V, D)(table.T, jnp.eye(D, dtype=jnp.float32))
    x2 = x.reshape(_NW, nchunk, _CHUNK)
    off_flat = np.tile(_OFFS, bpw // nf).reshape(nchunk, _CHUNK)
    out = _sc_gather(B, nchunk)(x2, jnp.asarray(off_flat), t2)
    return out.reshape(batch, nf, D)


# speedup vs baseline: 1.9580x; 1.0560x over previous
import functools

import jax
import jax.numpy as jnp
import numpy as np
from jax import lax
from jax.experimental import pallas as pl
from jax.experimental.pallas import tpu as pltpu
from jax.experimental.pallas import tpu_sc as plsc

_FIELD_DIMS = [100000] * 26
_EMBED_DIM = 64
_OFFS = np.array((0, *np.cumsum(_FIELD_DIMS)[:-1]), dtype=np.int32)

_NC = 2
_NS = 16
_NW = _NC * _NS
_LANES = 16
_CHUNK = 128
_CONV_COLS = 32768


_HALF = _CONV_COLS // 2
_SH = _CONV_COLS.bit_length() - 1


def _conv_body(tt_ref, eye_ref, out_ref):
    t = lax.dot_general(tt_ref[...], eye_ref[...],
                        (((0,), (0,)), ((), ())),
                        preferred_element_type=jnp.float32)
    out_ref[...] = jnp.concatenate([t[:_HALF], t[_HALF:]], axis=1)


@functools.lru_cache(maxsize=None)
def _conv_tc(V, D):
    grid = (V + _CONV_COLS - 1) // _CONV_COLS
    return pl.pallas_call(
        _conv_body,
        grid=(grid,),
        in_specs=[
            pl.BlockSpec((D, _CONV_COLS), lambda i: (0, i)),
            pl.BlockSpec((D, D), lambda i: (0, 0)),
        ],
        out_specs=pl.BlockSpec((_HALF, 2 * D), lambda i: (i, 0)),
        out_shape=jax.ShapeDtypeStruct((grid * _HALF, 2 * D), jnp.float32),
    )


@functools.lru_cache(maxsize=None)
def _sc_gather(B, nchunk):
    mesh = plsc.VectorSubcoreMesh(core_axis_name="c", subcore_axis_name="s")
    bpw = nchunk * _CHUNK

    @functools.partial(
        pl.kernel,
        mesh=mesh,
        out_type=jax.ShapeDtypeStruct((B // 2, 128), jnp.float32),
        scratch_types=[
            pltpu.VMEM((nchunk, _CHUNK), jnp.int32),
            pltpu.VMEM((nchunk, _CHUNK), jnp.int32),
            pltpu.VMEM((nchunk, _CHUNK), jnp.int32),
            pltpu.VMEM((nchunk, _CHUNK), jnp.int32),
            pltpu.VMEM((_CHUNK, 128), jnp.float32),
            pltpu.VMEM((_CHUNK, 128), jnp.float32),
            pltpu.VMEM((_CHUNK // 2, 128), jnp.float32),
            pltpu.SemaphoreType.DMA,
            pltpu.SemaphoreType.DMA,
            pltpu.SemaphoreType.DMA,
        ],
        compiler_params=pltpu.CompilerParams(needs_layout_passes=False),
    )
    def k(x_hbm, off_hbm, t2_hbm, out_hbm,
          x_v, off_v, row_v, sel_v, gb0, gb1, stage, g0, g1, osem):
        wid = lax.axis_index("s") * _NC + lax.axis_index("c")

        pltpu.sync_copy(x_hbm.at[wid], x_v)
        pltpu.sync_copy(off_hbm, off_v)

        def idx_body(c, carry):
            for j in range(_CHUNK // _LANES):
                sl = pl.ds(j * _LANES, _LANES)
                idx = x_v[c, sl] + off_v[c, sl]
                row_v[c, sl] = ((idx >> _SH) << (_SH - 1)) | (idx & (_HALF - 1))
                sel_v[c, sl] = ((idx >> (_SH - 1)) & 1) << 6
            return carry

        lax.fori_loop(0, nchunk, idx_body, 0)

        gbufs, gsems = (gb0, gb1), (g0, g1)

        def start_gather(c, b):
            pltpu.async_copy(t2_hbm.at[row_v.at[c]], gbufs[b], gsems[b])

        def wait_gather(c, b):
            pltpu.make_async_copy(t2_hbm.at[row_v.at[c]], gbufs[b],
                                  gsems[b]).wait()

        def out_slice(c):
            return out_hbm.at[pl.ds(wid * (bpw // 2) + c * (_CHUNK // 2),
                                    _CHUNK // 2)]

        iota = lax.iota(jnp.int32, _LANES)

        def select_chunk(c, b):
            gbuf = gbufs[b]

            def sel_body(kk, carry):
                selv = sel_v[c, pl.ds(kk * _LANES, _LANES)]
                for j in range(_LANES):
                    p = kk * _LANES + j
                    rowsplat = jnp.full((_LANES,), p, jnp.int32)
                    colbase = jnp.take(selv, jnp.full((_LANES,), j, jnp.int32))
                    for m in range(_EMBED_DIM // _LANES):
                        vals = plsc.load_gather(
                            gbuf, [rowsplat, colbase + (iota + m * _LANES)])
                        stage[p >> 1,
                              pl.ds((p & 1) * _EMBED_DIM + m * _LANES,
                                    _LANES)] = vals
                return carry

            lax.fori_loop(0, _CHUNK // _LANES, sel_body, 0)

        start_gather(0, 0)
        start_gather(1, 1)

        def pipe_body(h, carry):
            c = h * 2
            for b in range(2):
                cc = c + b
                wait_gather(cc, b)
                select_chunk(cc, b)
                start_gather(cc + 2, b)
                pltpu.sync_copy(stage, out_slice(cc))
            return carry

        lax.fori_loop(0, (nchunk - 2) // 2, pipe_body, 0)

        for b in range(2):
            cc = nchunk - 2 + b
            wait_gather(cc, b)
            select_chunk(cc, b)
            pltpu.sync_copy(stage, out_slice(cc))

    return k


def kernel(x, table):
    batch, nf = x.shape
    V, D = table.shape
    B = batch * nf
    bpw = B // _NW
    nchunk = bpw // _CHUNK
    assert bpw % _CHUNK == 0 and bpw % nf == 0 and nchunk % 2 == 0
    assert D == _EMBED_DIM and V % 2 == 0

    t2 = _conv_tc(V, D)(table.T, jnp.eye(D, dtype=jnp.float32))
    x2 = x.reshape(_NW, nchunk, _CHUNK)
    off_flat = np.tile(_OFFS, bpw // nf).reshape(nchunk, _CHUNK)
    out = _sc_gather(B, nchunk)(x2, jnp.asarray(off_flat), t2)
    return out.reshape(batch, nf, D)
